# Initial kernel scaffold; baseline (speedup 1.0000x reference)
import jax
import jax.numpy as jnp
from jax.experimental import pallas as pl


def kernel(input):
    raise NotImplementedError("write your pallas kernel here")


# speedup vs baseline: 341.8105x; 341.8105x over previous
import jax
import jax.numpy as jnp
from jax import lax
from jax.experimental import pallas as pl
from jax.experimental.pallas import tpu as pltpu

_BM = 512


def _groupsort2_block(x_ref, o_ref):
    x = x_ref[...]
    m, n = x.shape
    parity_even = (lax.broadcasted_iota(jnp.int32, (m, n), 1) & 1) == 0
    left = jnp.roll(x, -1, axis=1)
    right = jnp.roll(x, 1, axis=1)
    partner = jnp.where(parity_even, left, right)
    o_ref[...] = jnp.where(parity_even,
                           jnp.minimum(x, partner),
                           jnp.maximum(x, partner))


def kernel(input):
    m, n = input.shape
    grid = (m // _BM,)
    return pl.pallas_call(
        _groupsort2_block,
        grid=grid,
        in_specs=[pl.BlockSpec((_BM, n), lambda i: (i, 0))],
        out_specs=pl.BlockSpec((_BM, n), lambda i: (i, 0)),
        out_shape=jax.ShapeDtypeStruct((m, n), input.dtype),
    )(input)
